# baseline (speedup 1.0000x reference)
import functools

import jax
import jax.numpy as jnp
from jax.experimental import pallas as pl

N_ROWS = 8192
IN_DIM = 769
H = 64
A = 32
S = 64
OUT_W = A * S

TILE = 2048
GRID = N_ROWS // TILE


def _fused_body(x_ref, w1_ref, b1_ref, w2_ref, b2_ref, w3_ref, m4_ref, o_ref):
    xt = x_ref[...]
    g = xt[IN_DIM - 1:IN_DIM, :]
    ml = ((g >= 0.0) & (g < 0.5)).astype(jnp.float32)
    mm = ((g >= 0.5) & (g < 0.75)).astype(jnp.float32)
    mh = ((g >= 0.75) & (g <= 1.0)).astype(jnp.float32)

    h1 = jnp.dot(w1_ref[...], xt.astype(jnp.bfloat16),
                 preferred_element_type=jnp.float32) + b1_ref[...]
    h1 = jnp.maximum(h1, 0.0)
    h1m = jnp.concatenate(
        [h1[0:H] * ml, h1[H:2 * H] * mm, h1[2 * H:3 * H] * mh], axis=0)

    h2 = jnp.dot(w2_ref[...], h1m.astype(jnp.bfloat16),
                 preferred_element_type=jnp.float32) + b2_ref[...]
    h2 = jnp.maximum(h2, 0.0)

    zeros_pad = jnp.zeros((256 - 3 * H - 3, TILE), jnp.float32)
    aug = jnp.concatenate(
        [h2[0:H] * ml, h2[H:2 * H] * mm, h2[2 * H:3 * H] * mh,
         ml, mm, mh, zeros_pad], axis=0)

    y = jnp.dot(w3_ref[...], aug.astype(jnp.bfloat16),
                preferred_element_type=jnp.float32)
    y = jnp.maximum(y, 0.0)

    yb = y.astype(jnp.bfloat16)
    m4 = m4_ref[...]
    for m in range(8):
        o_ref[m * 256:(m + 1) * 256, :] = jnp.dot(
            m4, yb[m * 256:(m + 1) * 256, :],
            preferred_element_type=jnp.float32)


@functools.partial(jax.jit, static_argnames=())
def _prep_and_run(x, lW1, lb1, lW2, lb2, lW3, lb3,
                  mW1, mb1, mW2, mb2, mW3, mb3,
                  hW1, hb1, hW2, hb2, hW3, hb3):
    f32 = jnp.float32
    bf16 = jnp.bfloat16

    xT = x.T

    w1t = jnp.concatenate([lW1.T, mW1.T, hW1.T], axis=0).astype(bf16)
    b1t = jnp.concatenate([lb1, mb1, hb1]).reshape(3 * H, 1).astype(f32)

    zb = jnp.zeros((H, H), f32)
    w2t = jnp.block([[lW2.T, zb, zb], [zb, mW2.T, zb],
                     [zb, zb, hW2.T]]).astype(bf16)
    b2t = jnp.concatenate([lb2, mb2, hb2]).reshape(3 * H, 1).astype(f32)

    w3t = jnp.concatenate(
        [lW3.T, mW3.T, hW3.T, lb3.reshape(OUT_W, 1), mb3.reshape(OUT_W, 1),
         hb3.reshape(OUT_W, 1), jnp.zeros((OUT_W, 256 - 3 * H - 3), f32)],
        axis=1).astype(bf16)

    jj = jax.lax.broadcasted_iota(jnp.int32, (256, 256), 0)
    kk = jax.lax.broadcasted_iota(jnp.int32, (256, 256), 1)
    m4 = (((jj // S) == (kk // S)) & ((jj % S) + (kk % S) <= S - 1)).astype(bf16)

    out = pl.pallas_call(
        _fused_body,
        grid=(GRID,),
        in_specs=[
            pl.BlockSpec((IN_DIM, TILE), lambda t: (0, t)),
            pl.BlockSpec((3 * H, IN_DIM), lambda t: (0, 0)),
            pl.BlockSpec((3 * H, 1), lambda t: (0, 0)),
            pl.BlockSpec((3 * H, 3 * H), lambda t: (0, 0)),
            pl.BlockSpec((3 * H, 1), lambda t: (0, 0)),
            pl.BlockSpec((OUT_W, 256), lambda t: (0, 0)),
            pl.BlockSpec((256, 256), lambda t: (0, 0)),
        ],
        out_specs=pl.BlockSpec((OUT_W, TILE), lambda t: (0, t)),
        out_shape=jax.ShapeDtypeStruct((OUT_W, N_ROWS), f32),
    )(xT, w1t, b1t, w2t, b2t, w3t, m4)
    return out.reshape(A, S, N_ROWS).transpose(2, 0, 1)


def kernel(x, lW1, lb1, lW2, lb2, lW3, lb3, mW1, mb1, mW2, mb2, mW3, mb3,
           hW1, hb1, hW2, hb2, hW3, hb3):
    return _prep_and_run(x, lW1, lb1, lW2, lb2, lW3, lb3,
                         mW1, mb1, mW2, mb2, mW3, mb3,
                         hW1, hb1, hW2, hb2, hW3, hb3)


# speedup vs baseline: 1.0050x; 1.0050x over previous
import functools

import jax
import jax.numpy as jnp
from jax.experimental import pallas as pl

N_ROWS = 8192
IN_DIM = 769
H = 64
A = 32
S = 64
OUT_W = A * S

TILE = 1024
GRID = N_ROWS // TILE


def _fused_body(x_ref, w1_ref, b1_ref, w2_ref, b2_ref, w3_ref, m4_ref, o_ref):
    xt = x_ref[...]
    g = xt[IN_DIM - 1:IN_DIM, :]
    ml = ((g >= 0.0) & (g < 0.5)).astype(jnp.float32)
    mm = ((g >= 0.5) & (g < 0.75)).astype(jnp.float32)
    mh = ((g >= 0.75) & (g <= 1.0)).astype(jnp.float32)

    h1 = jax.lax.dot_general(
        w1_ref[...], xt.astype(jnp.bfloat16),
        dimension_numbers=(((0,), (0,)), ((), ())),
        preferred_element_type=jnp.float32) + b1_ref[...]
    h1 = jnp.maximum(h1, 0.0)
    h1m = jnp.concatenate(
        [h1[0:H] * ml, h1[H:2 * H] * mm, h1[2 * H:3 * H] * mh], axis=0)

    h2 = jax.lax.dot_general(
        w2_ref[...], h1m.astype(jnp.bfloat16),
        dimension_numbers=(((0,), (0,)), ((), ())),
        preferred_element_type=jnp.float32) + b2_ref[...]
    h2 = jnp.maximum(h2, 0.0)

    zeros_pad = jnp.zeros((256 - 3 * H - 3, TILE), jnp.float32)
    aug = jnp.concatenate(
        [h2[0:H] * ml, h2[H:2 * H] * mm, h2[2 * H:3 * H] * mh,
         ml, mm, mh, zeros_pad], axis=0)

    y = jax.lax.dot_general(
        w3_ref[...], aug.astype(jnp.bfloat16),
        dimension_numbers=(((0,), (0,)), ((), ())),
        preferred_element_type=jnp.float32)
    y = jnp.maximum(y, 0.0)

    yb = y.astype(jnp.bfloat16)
    m4 = m4_ref[...]
    for m in range(8):
        o_ref[m * 256:(m + 1) * 256, :] = jnp.dot(
            m4, yb[m * 256:(m + 1) * 256, :],
            preferred_element_type=jnp.float32)


@functools.partial(jax.jit, static_argnames=())
def _prep_and_run(x, lW1, lb1, lW2, lb2, lW3, lb3,
                  mW1, mb1, mW2, mb2, mW3, mb3,
                  hW1, hb1, hW2, hb2, hW3, hb3):
    f32 = jnp.float32
    bf16 = jnp.bfloat16

    xT = x.T

    w1t = jnp.concatenate([lW1, mW1, hW1], axis=1).astype(bf16)
    b1t = jnp.concatenate([lb1, mb1, hb1]).reshape(3 * H, 1).astype(f32)

    zb = jnp.zeros((H, H), f32)
    w2t = jnp.block([[lW2, zb, zb], [zb, mW2, zb],
                     [zb, zb, hW2]]).astype(bf16)
    b2t = jnp.concatenate([lb2, mb2, hb2]).reshape(3 * H, 1).astype(f32)

    w3t = jnp.concatenate(
        [lW3, mW3, hW3, lb3.reshape(1, OUT_W), mb3.reshape(1, OUT_W),
         hb3.reshape(1, OUT_W), jnp.zeros((256 - 3 * H - 3, OUT_W), f32)],
        axis=0).astype(bf16)

    jj = jax.lax.broadcasted_iota(jnp.int32, (256, 256), 0)
    kk = jax.lax.broadcasted_iota(jnp.int32, (256, 256), 1)
    m4 = (((jj // S) == (kk // S)) & ((jj % S) + (kk % S) <= S - 1)).astype(bf16)

    out = pl.pallas_call(
        _fused_body,
        grid=(GRID,),
        in_specs=[
            pl.BlockSpec((IN_DIM, TILE), lambda t: (0, t)),
            pl.BlockSpec((IN_DIM, 3 * H), lambda t: (0, 0)),
            pl.BlockSpec((3 * H, 1), lambda t: (0, 0)),
            pl.BlockSpec((3 * H, 3 * H), lambda t: (0, 0)),
            pl.BlockSpec((3 * H, 1), lambda t: (0, 0)),
            pl.BlockSpec((256, OUT_W), lambda t: (0, 0)),
            pl.BlockSpec((256, 256), lambda t: (0, 0)),
        ],
        out_specs=pl.BlockSpec((OUT_W, TILE), lambda t: (0, t)),
        out_shape=jax.ShapeDtypeStruct((OUT_W, N_ROWS), f32),
    )(xT, w1t, b1t, w2t, b2t, w3t, m4)
    return out.reshape(A, S, N_ROWS).transpose(2, 0, 1)


def kernel(x, lW1, lb1, lW2, lb2, lW3, lb3, mW1, mb1, mW2, mb2, mW3, mb3,
           hW1, hb1, hW2, hb2, hW3, hb3):
    return _prep_and_run(x, lW1, lb1, lW2, lb2, lW3, lb3,
                         mW1, mb1, mW2, mb2, mW3, mb3,
                         hW1, hb1, hW2, hb2, hW3, hb3)


# speedup vs baseline: 1.4151x; 1.4080x over previous
import functools

import jax
import jax.numpy as jnp
from jax.experimental import pallas as pl

N_ROWS = 8192
IN_DIM = 769
H = 64
A = 32
S = 64
OUT_W = A * S

TILE = 1024
GRID = N_ROWS // TILE


def _fused_body(x_ref, w1_ref, b1_ref, w2_ref, b2_ref, w3_ref, m4_ref, o_ref):
    xt = x_ref[...]
    g = xt[IN_DIM - 1:IN_DIM, :]
    ml = ((g >= 0.0) & (g < 0.5)).astype(jnp.float32)
    mm = ((g >= 0.5) & (g < 0.75)).astype(jnp.float32)
    mh = ((g >= 0.75) & (g <= 1.0)).astype(jnp.float32)

    h1 = jnp.dot(w1_ref[...], xt.astype(jnp.bfloat16),
                 preferred_element_type=jnp.float32) + b1_ref[...]
    h1 = jnp.maximum(h1, 0.0)
    h1m = jnp.concatenate(
        [h1[0:H] * ml, h1[H:2 * H] * mm, h1[2 * H:3 * H] * mh], axis=0)

    h2 = jax.lax.dot_general(
        w2_ref[...], h1m.astype(jnp.bfloat16),
        dimension_numbers=(((0,), (0,)), ((), ())),
        preferred_element_type=jnp.float32) + b2_ref[...]
    h2 = jnp.maximum(h2, 0.0)

    zeros_pad = jnp.zeros((256 - 3 * H - 3, TILE), jnp.float32)
    aug = jnp.concatenate(
        [h2[0:H] * ml, h2[H:2 * H] * mm, h2[2 * H:3 * H] * mh,
         ml, mm, mh, zeros_pad], axis=0)

    y = jax.lax.dot_general(
        w3_ref[...], aug.astype(jnp.bfloat16),
        dimension_numbers=(((0,), (0,)), ((), ())),
        preferred_element_type=jnp.float32)
    y = jnp.maximum(y, 0.0)

    yb = y.astype(jnp.bfloat16)
    m4 = m4_ref[...]
    for m in range(8):
        o_ref[m * 256:(m + 1) * 256, :] = jnp.dot(
            m4, yb[m * 256:(m + 1) * 256, :],
            preferred_element_type=jnp.float32)


def _prep_body(lw1_ref, mw1_ref, hw1_ref, lw2_ref, mw2_ref, hw2_ref,
               lw3_ref, mw3_ref, hw3_ref, lb3_ref, mb3_ref, hb3_ref,
               w1_ref, w2_ref, w3_ref, m4_ref):
    bf16 = jnp.bfloat16
    w1_ref[...] = jnp.concatenate(
        [lw1_ref[...], mw1_ref[...], hw1_ref[...]], axis=0).astype(bf16)
    z64 = jnp.zeros((H, H), jnp.float32)
    w2_ref[...] = jnp.concatenate([
        jnp.concatenate([lw2_ref[...], z64, z64], axis=1),
        jnp.concatenate([z64, mw2_ref[...], z64], axis=1),
        jnp.concatenate([z64, z64, hw2_ref[...]], axis=1)], axis=0).astype(bf16)
    w3_ref[...] = jnp.concatenate(
        [lw3_ref[...], mw3_ref[...], hw3_ref[...], lb3_ref[...], mb3_ref[...],
         hb3_ref[...], jnp.zeros((256 - 3 * H - 3, OUT_W), jnp.float32)],
        axis=0).astype(bf16)
    jj = jax.lax.broadcasted_iota(jnp.int32, (256, 256), 0)
    kk = jax.lax.broadcasted_iota(jnp.int32, (256, 256), 1)
    m4_ref[...] = (((jj // S) == (kk // S))
                   & ((jj % S) + (kk % S) <= S - 1)).astype(bf16)


@functools.partial(jax.jit, static_argnames=())
def _prep_and_run(x, lW1, lb1, lW2, lb2, lW3, lb3,
                  mW1, mb1, mW2, mb2, mW3, mb3,
                  hW1, hb1, hW2, hb2, hW3, hb3):
    f32 = jnp.float32
    bf16 = jnp.bfloat16

    xT = x.T
    b1t = jnp.concatenate([lb1, mb1, hb1]).reshape(3 * H, 1).astype(f32)
    b2t = jnp.concatenate([lb2, mb2, hb2]).reshape(3 * H, 1).astype(f32)

    full = lambda shp: pl.BlockSpec(shp, lambda: tuple(0 for _ in shp))
    w1t, w2t, w3t, m4 = pl.pallas_call(
        _prep_body,
        in_specs=[full((H, IN_DIM))] * 3 + [full((H, H))] * 3
                 + [full((H, OUT_W))] * 3 + [full((1, OUT_W))] * 3,
        out_specs=[full((3 * H, IN_DIM)), full((3 * H, 3 * H)),
                   full((256, OUT_W)), full((256, 256))],
        out_shape=[jax.ShapeDtypeStruct((3 * H, IN_DIM), bf16),
                   jax.ShapeDtypeStruct((3 * H, 3 * H), bf16),
                   jax.ShapeDtypeStruct((256, OUT_W), bf16),
                   jax.ShapeDtypeStruct((256, 256), bf16)],
    )(lW1.T, mW1.T, hW1.T, lW2, mW2, hW2, lW3, mW3, hW3,
      lb3.reshape(1, OUT_W), mb3.reshape(1, OUT_W), hb3.reshape(1, OUT_W))

    out = pl.pallas_call(
        _fused_body,
        grid=(GRID,),
        in_specs=[
            pl.BlockSpec((IN_DIM, TILE), lambda t: (0, t)),
            pl.BlockSpec((3 * H, IN_DIM), lambda t: (0, 0)),
            pl.BlockSpec((3 * H, 1), lambda t: (0, 0)),
            pl.BlockSpec((3 * H, 3 * H), lambda t: (0, 0)),
            pl.BlockSpec((3 * H, 1), lambda t: (0, 0)),
            pl.BlockSpec((256, OUT_W), lambda t: (0, 0)),
            pl.BlockSpec((256, 256), lambda t: (0, 0)),
        ],
        out_specs=pl.BlockSpec((OUT_W, TILE), lambda t: (0, t)),
        out_shape=jax.ShapeDtypeStruct((OUT_W, N_ROWS), f32),
    )(xT, w1t, b1t, w2t, b2t, w3t, m4)
    return out.reshape(A, S, N_ROWS).transpose(2, 0, 1)


def kernel(x, lW1, lb1, lW2, lb2, lW3, lb3, mW1, mb1, mW2, mb2, mW3, mb3,
           hW1, hb1, hW2, hb2, hW3, hb3):
    return _prep_and_run(x, lW1, lb1, lW2, lb2, lW3, lb3,
                         mW1, mb1, mW2, mb2, mW3, mb3,
                         hW1, hb1, hW2, hb2, hW3, hb3)
